# baseline (speedup 1.0000x reference)
import functools

import jax
import jax.numpy as jnp
from jax import lax
from jax.experimental import pallas as pl
from jax.experimental.pallas import tpu as pltpu
from jax.experimental.pallas import tpu_sc as plsc

N = 10000
E = 320000
D = 128

NC = 2
NS = 16
NW = NC * NS

CH = 128
E_PAD = 327680
CHUNKS = E_PAD // CH
CPT = CHUNKS // NW
CPG_A = 8
NG_A = CPT // CPG_A
CPG_B = 16
NG_B = CPT // CPG_B
N_PAD = 10240
ROWS_PER_TILE = N_PAD // NS

_mesh = plsc.VectorSubcoreMesh(core_axis_name="c", subcore_axis_name="s")


@functools.partial(
    pl.kernel,
    out_type=jax.ShapeDtypeStruct((NC, N_PAD, D), jnp.float32),
    mesh=_mesh,
    scratch_types=[
        pltpu.VMEM((CPG_A, CH), jnp.int32),
        pltpu.VMEM((CPG_A, CH), jnp.int32),
        pltpu.VMEM((2, CH, D), jnp.float32),
        pltpu.VMEM_SHARED((N_PAD, D), jnp.float32),
        pltpu.SemaphoreType.DMA,
        pltpu.SemaphoreType.DMA,
        pltpu.SemaphoreType.DMA,
        pltpu.SemaphoreType.DMA,
    ],
)
def _sc_sum(src2d, dst2d, x_hbm, zrow_hbm, out_sum,
            srcv, dstv, rows, acc_sh, sg0, sg1, sc0, sc1):
    c = lax.axis_index("c")
    s = lax.axis_index("s")
    wid = s * NC + c
    r0 = s * ROWS_PER_TILE
    sg = (sg0, sg1)
    sc = (sc0, sc1)

    pltpu.sync_copy(zrow_hbm, rows.at[0])
    for k in range(ROWS_PER_TILE // CH):
        pltpu.sync_copy(rows.at[0], acc_sh.at[pl.ds(r0 + k * CH, CH)])
    plsc.subcore_barrier()

    def _drain_sc(b):
        pltpu.make_async_copy(
            rows.at[b], acc_sh.at[dstv.at[CPG_A - 2 + b]], sc[b]).wait()

    def group(g, carry):
        base = wid * CPT + g * CPG_A

        @pl.when(g > 0)
        def _():
            _drain_sc(0)
            _drain_sc(1)

        pltpu.sync_copy(src2d.at[pl.ds(base, CPG_A)], srcv)
        pltpu.sync_copy(dst2d.at[pl.ds(base, CPG_A)], dstv)

        d0 = pltpu.async_copy(x_hbm.at[srcv.at[0]], rows.at[0], sg0)
        d1 = pltpu.async_copy(x_hbm.at[srcv.at[1]], rows.at[1], sg1)
        gd = [d0, d1]
        for i in range(CPG_A):
            b = i % 2
            gd[b].wait()
            sdesc = pltpu.async_copy(
                rows.at[b], acc_sh.at[dstv.at[i]], sc[b], add=True)
            if i + 2 < CPG_A:
                sdesc.wait()
                gd[b] = pltpu.async_copy(
                    x_hbm.at[srcv.at[i + 2]], rows.at[b], sg[b])
        return carry

    lax.fori_loop(0, NG_A, group, 0)
    _drain_sc(0)
    _drain_sc(1)
    plsc.subcore_barrier()

    for k in range(ROWS_PER_TILE // CH):
        pltpu.sync_copy(acc_sh.at[pl.ds(r0 + k * CH, CH)], rows.at[0])
        pltpu.sync_copy(rows.at[0], out_sum.at[c, pl.ds(r0 + k * CH, CH)])


@functools.partial(
    pl.kernel,
    out_type=jax.ShapeDtypeStruct((NC, N_PAD, D), jnp.float32),
    mesh=_mesh,
    scratch_types=[
        pltpu.VMEM((CPG_B, CH), jnp.int32),
        pltpu.VMEM((CH, D), jnp.float32),
        pltpu.VMEM_SHARED((N_PAD, D), jnp.float32),
        pltpu.SemaphoreType.DMA,
    ],
)
def _sc_count(dst2d, zrow_hbm, ones_hbm, out_cnt, dstv, ones_v, cnt_sh, sem):
    c = lax.axis_index("c")
    s = lax.axis_index("s")
    wid = s * NC + c
    r0 = s * ROWS_PER_TILE

    pltpu.sync_copy(zrow_hbm, ones_v)
    for k in range(ROWS_PER_TILE // CH):
        pltpu.sync_copy(ones_v, cnt_sh.at[pl.ds(r0 + k * CH, CH)])
    pltpu.sync_copy(ones_hbm, ones_v)
    plsc.subcore_barrier()

    def _drain_all():
        for i in range(CPG_B):
            pltpu.make_async_copy(ones_v, cnt_sh.at[dstv.at[i]], sem).wait()

    def group(g, carry):
        base = wid * CPT + g * CPG_B

        @pl.when(g > 0)
        def _():
            _drain_all()

        pltpu.sync_copy(dst2d.at[pl.ds(base, CPG_B)], dstv)
        for i in range(CPG_B):
            pltpu.async_copy(ones_v, cnt_sh.at[dstv.at[i]], sem, add=True)
        return carry

    lax.fori_loop(0, NG_B, group, 0)
    _drain_all()
    plsc.subcore_barrier()

    for k in range(ROWS_PER_TILE // CH):
        pltpu.sync_copy(cnt_sh.at[pl.ds(r0 + k * CH, CH)], ones_v)
        pltpu.sync_copy(ones_v, out_cnt.at[c, pl.ds(r0 + k * CH, CH)])


BLK = 400


def _tc_epilogue(ps_ref, cnt_ref, x_ref, wl_ref, wr_ref, bl_ref, o_ref):
    p = ps_ref[0] + ps_ref[1]
    cnt = (cnt_ref[0] + cnt_ref[1])[:, :1]
    mean = p / jnp.clip(cnt, 1.0, None)
    acc = jnp.dot(mean, wl_ref[...], preferred_element_type=jnp.float32)
    acc = acc + jnp.dot(x_ref[...], wr_ref[...],
                        preferred_element_type=jnp.float32)
    o_ref[...] = jnp.maximum(acc + bl_ref[...], 0.0)


def kernel(x, edge_index, W_l, W_r, b_l):
    dst = edge_index[0].astype(jnp.int32)
    src = edge_index[1].astype(jnp.int32)
    npad = E_PAD - E
    src2d = jnp.concatenate(
        [src, jnp.zeros((npad,), jnp.int32)]).reshape(CHUNKS, CH)
    dst2d = jnp.concatenate(
        [dst, jnp.full((npad,), N, jnp.int32)]).reshape(CHUNKS, CH)
    zrow = jnp.zeros((CH, D), jnp.float32)
    ones = jnp.ones((CH, D), jnp.float32)

    psum = _sc_sum(src2d, dst2d, x, zrow)
    pcnt = _sc_count(dst2d, zrow, ones)

    out = pl.pallas_call(
        _tc_epilogue,
        grid=(N // BLK,),
        in_specs=[
            pl.BlockSpec((NC, BLK, D), lambda i: (0, i, 0)),
            pl.BlockSpec((NC, BLK, D), lambda i: (0, i, 0)),
            pl.BlockSpec((BLK, D), lambda i: (i, 0)),
            pl.BlockSpec((D, D), lambda i: (0, 0)),
            pl.BlockSpec((D, D), lambda i: (0, 0)),
            pl.BlockSpec((1, D), lambda i: (0, 0)),
        ],
        out_specs=pl.BlockSpec((BLK, D), lambda i: (i, 0)),
        out_shape=jax.ShapeDtypeStruct((N, D), jnp.float32),
    )(psum, pcnt, x, W_l, W_r, b_l.reshape(1, D))
    return out
